# baseline (speedup 1.0000x reference)
import jax
import jax.numpy as jnp
from jax.experimental import pallas as pl
from jax.experimental.pallas import tpu as pltpu

N = 4096
N_SPECIES = 16
D_EMBED = 32
N_BASIS = 8
K_NEIGH = 32
R_MAX = 5.0
NM_TO_ANG = 10.0
ENERGY_TO_KJ = 96.48533212331

QBLK = 512
GRID = N // QBLK


def _mace_kernel(pos_ref, posT_ref, attrs_ref, wemb_ref, wmsg_ref, wout_ref,
                 out_ref, h_ref):
    step = pl.program_id(0)

    q = pos_ref[...] * NM_TO_ANG
    kT = posT_ref[...] * NM_TO_ANG
    d2 = None
    for c in range(3):
        diff = q[:, c:c + 1] - kT[c:c + 1, :]
        d2 = diff * diff if d2 is None else d2 + diff * diff

    row_g = jax.lax.broadcasted_iota(jnp.int32, (QBLK, N), 0) + step * QBLK
    col_g = jax.lax.broadcasted_iota(jnp.int32, (QBLK, N), 1)
    d2 = jnp.where(row_g == col_g, d2 + 1e10, d2)

    lo = jnp.zeros((QBLK, 1), jnp.float32)
    hi = jnp.full((QBLK, 1), R_MAX * R_MAX, jnp.float32)
    for _ in range(18):
        mid = 0.5 * (lo + hi)
        cnt = jnp.sum(jnp.where(d2 <= mid, 1.0, 0.0), axis=1, keepdims=True)
        take_hi = cnt >= K_NEIGH
        hi = jnp.where(take_hi, mid, hi)
        lo = jnp.where(take_hi, lo, mid)
    hi = jnp.max(jnp.where(d2 <= hi, d2, -1.0), axis=1, keepdims=True)
    sel = d2 <= hi

    @pl.when(step == 0)
    def _():
        h_ref[...] = jnp.dot(attrs_ref[...].astype(jnp.bfloat16),
                             wemb_ref[...].astype(jnp.bfloat16),
                             preferred_element_type=jnp.float32)
        out_ref[...] = jnp.zeros((1, 1), jnp.float32)

    z = d2 + 1e-12
    r = jax.lax.rsqrt(z)
    r = r * (1.5 - 0.5 * z * r * r)
    inv_dist = r * (1.5 - 0.5 * z * r * r)
    dist = z * inv_dist
    theta = jnp.minimum((jnp.pi / R_MAX) * dist, jnp.float32(jnp.pi))
    phi = theta - jnp.float32(jnp.pi / 2)
    p2 = phi * phi
    s_cos = 1.0 + p2 * (-1 / 2 + p2 * (1 / 24 + p2 * (-1 / 720 + p2 * (
        1 / 40320 + p2 * (-1 / 3628800 + p2 * (1 / 479001600))))))
    s_sin = phi * (1.0 + p2 * (-1 / 6 + p2 * (1 / 120 + p2 * (-1 / 5040 + p2 * (
        1 / 362880 + p2 * (-1 / 39916800 + p2 * (1 / 6227020800)))))))
    s_b = s_cos
    cw = -s_sin
    fc = 0.5 * (cw + 1.0)
    scale = jnp.where(sel, fc * inv_dist, 0.0)
    two_c = 2.0 * cw

    h_all = h_ref[...]
    h_hi = h_all.astype(jnp.bfloat16)
    h_lo = (h_all - h_hi.astype(jnp.float32)).astype(jnp.bfloat16)
    wmsg_b16 = wmsg_ref[...].astype(jnp.bfloat16).astype(jnp.float32)

    m = jnp.zeros((QBLK, D_EMBED), jnp.float32)
    s_prev = None
    for b in range(N_BASIS):
        if b == 1:
            s_prev, s_b = s_b, two_c * s_b
        elif b >= 2:
            s_prev, s_b = s_b, two_c * s_b - s_prev
        g = (s_b * scale).astype(jnp.bfloat16)
        p = (jnp.dot(g, h_hi, preferred_element_type=jnp.float32)
             + jnp.dot(g, h_lo, preferred_element_type=jnp.float32))
        m = m + p * wmsg_b16[b:b + 1, :]

    h_blk = h_ref[pl.ds(step * QBLK, QBLK), :]
    hm = (h_blk + m).astype(jnp.bfloat16).astype(jnp.float32)
    wout_b16 = wout_ref[...].astype(jnp.bfloat16).astype(jnp.float32)
    e_blk = jnp.sum(hm * wout_b16, axis=(0, 1), keepdims=True)
    out_ref[...] += e_blk * ENERGY_TO_KJ


@jax.jit
def kernel(positions, node_attrs, W_embed, W_msg, W_out):
    pos_pad = jnp.zeros((N, 8), jnp.float32).at[:, :3].set(positions)
    posT_pad = jnp.zeros((8, N), jnp.float32).at[:3, :].set(positions.T)
    wout2d = W_out.reshape(1, D_EMBED)

    out = pl.pallas_call(
        _mace_kernel,
        grid=(GRID,),
        in_specs=[
            pl.BlockSpec((QBLK, 8), lambda i: (i, 0)),
            pl.BlockSpec((8, N), lambda i: (0, 0)),
            pl.BlockSpec((N, N_SPECIES), lambda i: (0, 0)),
            pl.BlockSpec((N_SPECIES, D_EMBED), lambda i: (0, 0)),
            pl.BlockSpec((N_BASIS, D_EMBED), lambda i: (0, 0)),
            pl.BlockSpec((1, D_EMBED), lambda i: (0, 0)),
        ],
        out_specs=pl.BlockSpec((1, 1), lambda i: (0, 0)),
        out_shape=jax.ShapeDtypeStruct((1, 1), jnp.float32),
        scratch_shapes=[pltpu.VMEM((N, D_EMBED), jnp.float32)],
    )(pos_pad, posT_pad, node_attrs, W_embed, W_msg, wout2d)
    return out.reshape(())


# speedup vs baseline: 1.1421x; 1.1421x over previous
import jax
import jax.numpy as jnp
from jax.experimental import pallas as pl
from jax.experimental.pallas import tpu as pltpu

N = 4096
N_SPECIES = 16
D_EMBED = 32
N_BASIS = 8
K_NEIGH = 32
R_MAX = 5.0
NM_TO_ANG = 10.0
ENERGY_TO_KJ = 96.48533212331

QBLK = 256
GRID = N // QBLK


def _mace_kernel(pos_ref, posT_ref, attrs_ref, wemb_ref, wmsg_ref, wout_ref,
                 out_ref, h_ref):
    step = pl.program_id(0)

    q = pos_ref[...] * NM_TO_ANG
    kT = posT_ref[...] * NM_TO_ANG
    d2 = None
    for c in range(3):
        diff = q[:, c:c + 1] - kT[c:c + 1, :]
        d2 = diff * diff if d2 is None else d2 + diff * diff

    row_g = jax.lax.broadcasted_iota(jnp.int32, (QBLK, N), 0) + step * QBLK
    col_g = jax.lax.broadcasted_iota(jnp.int32, (QBLK, N), 1)
    d2 = jnp.where(row_g == col_g, d2 + 1e10, d2)

    lo = jnp.zeros((QBLK, 1), jnp.float32)
    hi = jnp.full((QBLK, 1), R_MAX * R_MAX, jnp.float32)
    for _ in range(18):
        mid = 0.5 * (lo + hi)
        cnt = jnp.sum(jnp.where(d2 <= mid, 1.0, 0.0), axis=1, keepdims=True)
        take_hi = cnt >= K_NEIGH
        hi = jnp.where(take_hi, mid, hi)
        lo = jnp.where(take_hi, lo, mid)
    hi = jnp.max(jnp.where(d2 <= hi, d2, -1.0), axis=1, keepdims=True)
    sel = d2 <= hi

    @pl.when(step == 0)
    def _():
        h_ref[...] = jnp.dot(attrs_ref[...].astype(jnp.bfloat16),
                             wemb_ref[...].astype(jnp.bfloat16),
                             preferred_element_type=jnp.float32)
        out_ref[...] = jnp.zeros((1, 1), jnp.float32)

    z = d2 + 1e-12
    r = jax.lax.rsqrt(z)
    r = r * (1.5 - 0.5 * z * r * r)
    inv_dist = r * (1.5 - 0.5 * z * r * r)
    dist = z * inv_dist
    theta = jnp.minimum((jnp.pi / R_MAX) * dist, jnp.float32(jnp.pi))
    phi = theta - jnp.float32(jnp.pi / 2)
    p2 = phi * phi
    s_cos = 1.0 + p2 * (-1 / 2 + p2 * (1 / 24 + p2 * (-1 / 720 + p2 * (
        1 / 40320 + p2 * (-1 / 3628800 + p2 * (1 / 479001600))))))
    s_sin = phi * (1.0 + p2 * (-1 / 6 + p2 * (1 / 120 + p2 * (-1 / 5040 + p2 * (
        1 / 362880 + p2 * (-1 / 39916800 + p2 * (1 / 6227020800)))))))
    s_b = s_cos
    cw = -s_sin
    fc = 0.5 * (cw + 1.0)
    scale = jnp.where(sel, fc * inv_dist, 0.0)
    two_c = 2.0 * cw

    h_all = h_ref[...]
    h_hi = h_all.astype(jnp.bfloat16)
    h_lo = (h_all - h_hi.astype(jnp.float32)).astype(jnp.bfloat16)
    wmsg_b16 = wmsg_ref[...].astype(jnp.bfloat16).astype(jnp.float32)

    m = jnp.zeros((QBLK, D_EMBED), jnp.float32)
    s_prev = None
    for b in range(N_BASIS):
        if b == 1:
            s_prev, s_b = s_b, two_c * s_b
        elif b >= 2:
            s_prev, s_b = s_b, two_c * s_b - s_prev
        g = (s_b * scale).astype(jnp.bfloat16)
        p = (jnp.dot(g, h_hi, preferred_element_type=jnp.float32)
             + jnp.dot(g, h_lo, preferred_element_type=jnp.float32))
        m = m + p * wmsg_b16[b:b + 1, :]

    h_blk = h_ref[pl.ds(step * QBLK, QBLK), :]
    hm = (h_blk + m).astype(jnp.bfloat16).astype(jnp.float32)
    wout_b16 = wout_ref[...].astype(jnp.bfloat16).astype(jnp.float32)
    e_blk = jnp.sum(hm * wout_b16, axis=(0, 1), keepdims=True)
    out_ref[...] += e_blk * ENERGY_TO_KJ


@jax.jit
def kernel(positions, node_attrs, W_embed, W_msg, W_out):
    pos_pad = jnp.zeros((N, 8), jnp.float32).at[:, :3].set(positions)
    posT_pad = jnp.zeros((8, N), jnp.float32).at[:3, :].set(positions.T)
    wout2d = W_out.reshape(1, D_EMBED)

    out = pl.pallas_call(
        _mace_kernel,
        grid=(GRID,),
        in_specs=[
            pl.BlockSpec((QBLK, 8), lambda i: (i, 0)),
            pl.BlockSpec((8, N), lambda i: (0, 0)),
            pl.BlockSpec((N, N_SPECIES), lambda i: (0, 0)),
            pl.BlockSpec((N_SPECIES, D_EMBED), lambda i: (0, 0)),
            pl.BlockSpec((N_BASIS, D_EMBED), lambda i: (0, 0)),
            pl.BlockSpec((1, D_EMBED), lambda i: (0, 0)),
        ],
        out_specs=pl.BlockSpec((1, 1), lambda i: (0, 0)),
        out_shape=jax.ShapeDtypeStruct((1, 1), jnp.float32),
        scratch_shapes=[pltpu.VMEM((N, D_EMBED), jnp.float32)],
    )(pos_pad, posT_pad, node_attrs, W_embed, W_msg, wout2d)
    return out.reshape(())
